# Initial kernel scaffold; baseline (speedup 1.0000x reference)
import jax
import jax.numpy as jnp
from jax.experimental import pallas as pl


def kernel(x0, edge_index, Wr1, Ws1, b1, Wr2, Ws2, b2, Wr3, Ws3, b3, Wl, bl):
    raise NotImplementedError("write your pallas kernel here")


# speedup vs baseline: 2.1391x; 2.1391x over previous
import jax
import jax.numpy as jnp
from jax import lax
from jax.experimental import pallas as pl
from jax.experimental.pallas import tpu as pltpu
from jax.experimental.pallas import tpu_sc as plsc

_N = 10000
_E = 320000
_D = 128
_C = 40
_NC = 2
_NS = 16
_HALF = _N // _NC
_ACC = 5120
_EPAD = 327680
_TCH = _EPAD // _NS // 128
_BLK = 1000


def _sc_segment_sum(x, src_r, dst_r, zeros_blk):
  mesh = plsc.VectorSubcoreMesh(core_axis_name="c", subcore_axis_name="s")

  def body(x_hbm, src_hbm, dst_hbm, zer_hbm, out_hbm,
           src_v, dstl_v, stage0, stage1, buf_v, acc_sh, sem0, sem1):
    c = lax.axis_index("c")
    s = lax.axis_index("s")
    iota = lax.iota(jnp.int32, 16)

    pltpu.sync_copy(src_hbm.at[s], src_v)
    pltpu.sync_copy(dst_hbm.at[s], dstl_v)

    pltpu.sync_copy(zer_hbm, buf_v)
    for k in range(40):
      @pl.when((k % _NS) == s)
      def _():
        pltpu.sync_copy(buf_v, acc_sh.at[pl.ds(k * 128, 128)])

    base = c * _HALF

    def trans(i, carry):
      for j in range(8):
        dv = dstl_v[i, pl.ds(j * 16, 16)]
        local = dv - base
        ok = (local >= 0) & (local < _HALF)
        dstl_v[i, pl.ds(j * 16, 16)] = jnp.where(
            ok, local, _HALF + ((j * 16) & 63) + iota)
      return carry

    lax.fori_loop(0, _TCH, trans, 0)
    plsc.subcore_barrier()

    def gather(j, stage, sem):
      return pltpu.async_copy(x_hbm.at[src_v.at[j]], stage, sem)

    gather(0, stage0, sem0)
    gather(1, stage1, sem1)

    def step(i, carry):
      j0 = 2 * i
      pltpu.make_async_copy(x_hbm.at[src_v.at[0]], stage0, sem0).wait()
      pltpu.sync_copy(stage0, acc_sh.at[dstl_v.at[j0]], add=True)

      @pl.when(j0 + 2 < _TCH)
      def _():
        gather(j0 + 2, stage0, sem0)

      pltpu.make_async_copy(x_hbm.at[src_v.at[0]], stage1, sem1).wait()
      pltpu.sync_copy(stage1, acc_sh.at[dstl_v.at[j0 + 1]], add=True)

      @pl.when(j0 + 3 < _TCH)
      def _():
        gather(j0 + 3, stage1, sem1)

      return carry

    lax.fori_loop(0, _TCH // 2, step, 0)
    plsc.subcore_barrier()

    for k in range(40):
      @pl.when((k % _NS) == s)
      def _():
        pltpu.sync_copy(acc_sh.at[pl.ds(k * 128, 128)], buf_v)
        pltpu.sync_copy(buf_v, out_hbm.at[pl.ds(c * 5120 + k * 128, 128)])

  f = pl.kernel(
      body,
      out_type=jax.ShapeDtypeStruct((2 * 5120, _D), jnp.float32),
      mesh=mesh,
      scratch_types=[
          pltpu.VMEM((_TCH, 128), jnp.int32),
          pltpu.VMEM((_TCH, 128), jnp.int32),
          pltpu.VMEM((128, _D), jnp.float32),
          pltpu.VMEM((128, _D), jnp.float32),
          pltpu.VMEM((128, _D), jnp.float32),
          pltpu.VMEM_SHARED((_ACC, _D), jnp.float32),
          pltpu.SemaphoreType.DMA,
          pltpu.SemaphoreType.DMA,
      ],
  )
  return f(x, src_r, dst_r, zeros_blk)


def _tc_layer(agg, x, WrT, WsT, br):
  def body(a_ref, x_ref, wr_ref, ws_ref, b_ref, o_ref):
    h = jnp.dot(a_ref[...], wr_ref[...], preferred_element_type=jnp.float32)
    h = h + jnp.dot(x_ref[...], ws_ref[...], preferred_element_type=jnp.float32)
    o_ref[...] = jnp.maximum(h + b_ref[...], 0.0)

  return pl.pallas_call(
      body,
      grid=(_N // _BLK,),
      in_specs=[
          pl.BlockSpec((_BLK, _D), lambda i: (i, 0)),
          pl.BlockSpec((_BLK, _D), lambda i: (i, 0)),
          pl.BlockSpec((_D, _D), lambda i: (0, 0)),
          pl.BlockSpec((_D, _D), lambda i: (0, 0)),
          pl.BlockSpec((1, _D), lambda i: (0, 0)),
      ],
      out_specs=pl.BlockSpec((_BLK, _D), lambda i: (i, 0)),
      out_shape=jax.ShapeDtypeStruct((_N, _D), jnp.float32),
  )(agg, x, WrT, WsT, br)


def _tc_final(agg, x2, Wr3T, Ws3T, b3r, x1, W1T, W2T, W3T, blr):
  def body(a_ref, x2_ref, wr_ref, ws_ref, b3_ref,
           x1_ref, w1_ref, w2_ref, w3_ref, bl_ref, o_ref):
    x3 = jnp.dot(a_ref[...], wr_ref[...], preferred_element_type=jnp.float32)
    x3 = x3 + jnp.dot(x2_ref[...], ws_ref[...],
                      preferred_element_type=jnp.float32)
    x3 = jnp.maximum(x3 + b3_ref[...], 0.0)
    logits = jnp.dot(x1_ref[...], w1_ref[...],
                     preferred_element_type=jnp.float32)
    logits = logits + jnp.dot(x2_ref[...], w2_ref[...],
                              preferred_element_type=jnp.float32)
    logits = logits + jnp.dot(x3, w3_ref[...],
                              preferred_element_type=jnp.float32)
    logits = logits + bl_ref[...]
    m = jnp.max(logits, axis=1, keepdims=True)
    z = logits - m
    lse = jnp.log(jnp.sum(jnp.exp(z), axis=1, keepdims=True))
    o_ref[...] = z - lse

  return pl.pallas_call(
      body,
      grid=(_N // _BLK,),
      in_specs=[
          pl.BlockSpec((_BLK, _D), lambda i: (i, 0)),
          pl.BlockSpec((_BLK, _D), lambda i: (i, 0)),
          pl.BlockSpec((_D, _D), lambda i: (0, 0)),
          pl.BlockSpec((_D, _D), lambda i: (0, 0)),
          pl.BlockSpec((1, _D), lambda i: (0, 0)),
          pl.BlockSpec((_BLK, _D), lambda i: (i, 0)),
          pl.BlockSpec((_D, 128), lambda i: (0, 0)),
          pl.BlockSpec((_D, 128), lambda i: (0, 0)),
          pl.BlockSpec((_D, 128), lambda i: (0, 0)),
          pl.BlockSpec((1, 128), lambda i: (0, 0)),
      ],
      out_specs=pl.BlockSpec((_BLK, 128), lambda i: (i, 0)),
      out_shape=jax.ShapeDtypeStruct((_N, 128), jnp.float32),
  )(agg, x2, Wr3T, Ws3T, b3r, x1, W1T, W2T, W3T, blr)


def kernel(x0, edge_index, Wr1, Ws1, b1, Wr2, Ws2, b2, Wr3, Ws3, b3, Wl, bl):
  npad = _EPAD - _E
  src_r = jnp.concatenate(
      [edge_index[0], jnp.zeros((npad,), jnp.int32)]).reshape(_NS, _TCH, 128)
  dst_r = jnp.concatenate(
      [edge_index[1], jnp.full((npad,), _N, jnp.int32)]).reshape(
          _NS, _TCH, 128)
  zeros_blk = jnp.zeros((128, _D), jnp.float32)

  Wr1T, Ws1T = Wr1.T, Ws1.T
  Wr2T, Ws2T = Wr2.T, Ws2.T
  Wr3T, Ws3T = Wr3.T, Ws3.T
  b1r = b1.reshape(1, _D)
  b2r = b2.reshape(1, _D)
  b3r = b3.reshape(1, _D)
  WlTp = jnp.pad(Wl.T, ((0, 0), (0, 128 - _C)))
  W1T, W2T, W3T = WlTp[:_D], WlTp[_D:2 * _D], WlTp[2 * _D:]
  blr = jnp.pad(bl, (0, 128 - _C), constant_values=-1e30).reshape(1, 128)

  def seg(x):
    parts = _sc_segment_sum(x, src_r, dst_r, zeros_blk)
    return jnp.concatenate([parts[:_HALF], parts[5120:5120 + _HALF]], axis=0)

  x1 = _tc_layer(seg(x0), x0, Wr1T, Ws1T, b1r)
  x2 = _tc_layer(seg(x1), x1, Wr2T, Ws2T, b2r)
  out = _tc_final(seg(x2), x2, Wr3T, Ws3T, b3r, x1, W1T, W2T, W3T, blr)
  return out[:, :_C]


# speedup vs baseline: 2.1795x; 1.0189x over previous
import jax
import jax.numpy as jnp
from jax import lax
from jax.experimental import pallas as pl
from jax.experimental.pallas import tpu as pltpu
from jax.experimental.pallas import tpu_sc as plsc

_N = 10000
_E = 320000
_D = 128
_C = 40
_NC = 2
_NS = 16
_HALF = _N // _NC
_ACC = 5120
_EPAD = 327680
_TCH = _EPAD // _NS // 128
_BLK = 1000


def _sc_segment_sum(x, src_r, dst_r, zeros_blk):
  mesh = plsc.VectorSubcoreMesh(core_axis_name="c", subcore_axis_name="s")

  def body(x_hbm, src_hbm, dst_hbm, zer_hbm, out_hbm,
           src_v, dstl_v, stage_v, acc_sh, gsem, ssem):
    c = lax.axis_index("c")
    s = lax.axis_index("s")
    iota = lax.iota(jnp.int32, 16)

    pltpu.sync_copy(src_hbm.at[s], src_v)
    pltpu.sync_copy(dst_hbm.at[s], dstl_v)

    pltpu.sync_copy(zer_hbm, stage_v.at[pl.ds(0, 128)])
    for k in range(40):
      @pl.when((k % _NS) == s)
      def _():
        pltpu.sync_copy(stage_v.at[pl.ds(0, 128)],
                        acc_sh.at[pl.ds(k * 128, 128)])

    base = c * _HALF

    def trans(i, carry):
      for j in range(8):
        dv = dstl_v[i, pl.ds(j * 16, 16)]
        local = dv - base
        ok = (local >= 0) & (local < _HALF)
        dstl_v[i, pl.ds(j * 16, 16)] = jnp.where(
            ok, local, _HALF + ((j * 16) & 63) + iota)
      return carry

    lax.fori_loop(0, _TCH, trans, 0)
    plsc.subcore_barrier()

    _NB = 3

    def stg(p):
      return stage_v.at[pl.ds(p * 128, 128)]

    def gather(j, p):
      return pltpu.async_copy(x_hbm.at[src_v.at[j]], stg(p), gsem.at[p])

    def wait_gather(p):
      pltpu.make_async_copy(x_hbm.at[src_v.at[0]], stg(p), gsem.at[p]).wait()

    def scatter(j, p):
      return pltpu.async_copy(stg(p), acc_sh.at[dstl_v.at[j]], ssem.at[p],
                              add=True)

    def wait_scatter(p):
      pltpu.make_async_copy(stg(p), acc_sh.at[dstl_v.at[0]],
                            ssem.at[p]).wait()

    def prime(p, carry):
      gather(p, p)
      return carry

    lax.fori_loop(0, _NB, prime, 0)

    def step(j, carry):
      p = lax.rem(j, _NB)
      wait_gather(p)
      scatter(j, p)

      @pl.when(j + _NB < _TCH)
      def _():
        wait_scatter(p)
        gather(j + _NB, p)

      return carry

    lax.fori_loop(0, _TCH, step, 0)

    def drain(p, carry):
      wait_scatter(p)
      return carry

    lax.fori_loop(0, _NB, drain, 0)
    plsc.subcore_barrier()

    for k in range(40):
      @pl.when((k % _NS) == s)
      def _():
        pltpu.sync_copy(acc_sh.at[pl.ds(k * 128, 128)],
                        stage_v.at[pl.ds(0, 128)])
        pltpu.sync_copy(stage_v.at[pl.ds(0, 128)],
                        out_hbm.at[pl.ds(c * 5120 + k * 128, 128)])

  f = pl.kernel(
      body,
      out_type=jax.ShapeDtypeStruct((2 * 5120, _D), jnp.float32),
      mesh=mesh,
      scratch_types=[
          pltpu.VMEM((_TCH, 128), jnp.int32),
          pltpu.VMEM((_TCH, 128), jnp.int32),
          pltpu.VMEM((3 * 128, _D), jnp.float32),
          pltpu.VMEM_SHARED((_ACC, _D), jnp.float32),
          pltpu.SemaphoreType.DMA((3,)),
          pltpu.SemaphoreType.DMA((3,)),
      ],
  )
  return f(x, src_r, dst_r, zeros_blk)


def _tc_layer(agg, x, WrT, WsT, br):
  def body(a_ref, x_ref, wr_ref, ws_ref, b_ref, o_ref):
    h = jnp.dot(a_ref[...], wr_ref[...], preferred_element_type=jnp.float32)
    h = h + jnp.dot(x_ref[...], ws_ref[...], preferred_element_type=jnp.float32)
    o_ref[...] = jnp.maximum(h + b_ref[...], 0.0)

  return pl.pallas_call(
      body,
      grid=(_N // _BLK,),
      in_specs=[
          pl.BlockSpec((_BLK, _D), lambda i: (i, 0)),
          pl.BlockSpec((_BLK, _D), lambda i: (i, 0)),
          pl.BlockSpec((_D, _D), lambda i: (0, 0)),
          pl.BlockSpec((_D, _D), lambda i: (0, 0)),
          pl.BlockSpec((1, _D), lambda i: (0, 0)),
      ],
      out_specs=pl.BlockSpec((_BLK, _D), lambda i: (i, 0)),
      out_shape=jax.ShapeDtypeStruct((_N, _D), jnp.float32),
  )(agg, x, WrT, WsT, br)


def _tc_final(agg, x2, Wr3T, Ws3T, b3r, x1, W1T, W2T, W3T, blr):
  def body(a_ref, x2_ref, wr_ref, ws_ref, b3_ref,
           x1_ref, w1_ref, w2_ref, w3_ref, bl_ref, o_ref):
    x3 = jnp.dot(a_ref[...], wr_ref[...], preferred_element_type=jnp.float32)
    x3 = x3 + jnp.dot(x2_ref[...], ws_ref[...],
                      preferred_element_type=jnp.float32)
    x3 = jnp.maximum(x3 + b3_ref[...], 0.0)
    logits = jnp.dot(x1_ref[...], w1_ref[...],
                     preferred_element_type=jnp.float32)
    logits = logits + jnp.dot(x2_ref[...], w2_ref[...],
                              preferred_element_type=jnp.float32)
    logits = logits + jnp.dot(x3, w3_ref[...],
                              preferred_element_type=jnp.float32)
    logits = logits + bl_ref[...]
    m = jnp.max(logits, axis=1, keepdims=True)
    z = logits - m
    lse = jnp.log(jnp.sum(jnp.exp(z), axis=1, keepdims=True))
    o_ref[...] = z - lse

  return pl.pallas_call(
      body,
      grid=(_N // _BLK,),
      in_specs=[
          pl.BlockSpec((_BLK, _D), lambda i: (i, 0)),
          pl.BlockSpec((_BLK, _D), lambda i: (i, 0)),
          pl.BlockSpec((_D, _D), lambda i: (0, 0)),
          pl.BlockSpec((_D, _D), lambda i: (0, 0)),
          pl.BlockSpec((1, _D), lambda i: (0, 0)),
          pl.BlockSpec((_BLK, _D), lambda i: (i, 0)),
          pl.BlockSpec((_D, 128), lambda i: (0, 0)),
          pl.BlockSpec((_D, 128), lambda i: (0, 0)),
          pl.BlockSpec((_D, 128), lambda i: (0, 0)),
          pl.BlockSpec((1, 128), lambda i: (0, 0)),
      ],
      out_specs=pl.BlockSpec((_BLK, 128), lambda i: (i, 0)),
      out_shape=jax.ShapeDtypeStruct((_N, 128), jnp.float32),
  )(agg, x2, Wr3T, Ws3T, b3r, x1, W1T, W2T, W3T, blr)


def kernel(x0, edge_index, Wr1, Ws1, b1, Wr2, Ws2, b2, Wr3, Ws3, b3, Wl, bl):
  npad = _EPAD - _E
  src_r = jnp.concatenate(
      [edge_index[0], jnp.zeros((npad,), jnp.int32)]).reshape(_NS, _TCH, 128)
  dst_r = jnp.concatenate(
      [edge_index[1], jnp.full((npad,), _N, jnp.int32)]).reshape(
          _NS, _TCH, 128)
  zeros_blk = jnp.zeros((128, _D), jnp.float32)

  Wr1T, Ws1T = Wr1.T, Ws1.T
  Wr2T, Ws2T = Wr2.T, Ws2.T
  Wr3T, Ws3T = Wr3.T, Ws3.T
  b1r = b1.reshape(1, _D)
  b2r = b2.reshape(1, _D)
  b3r = b3.reshape(1, _D)
  WlTp = jnp.pad(Wl.T, ((0, 0), (0, 128 - _C)))
  W1T, W2T, W3T = WlTp[:_D], WlTp[_D:2 * _D], WlTp[2 * _D:]
  blr = jnp.pad(bl, (0, 128 - _C), constant_values=-1e30).reshape(1, 128)

  def seg(x):
    parts = _sc_segment_sum(x, src_r, dst_r, zeros_blk)
    return jnp.concatenate([parts[:_HALF], parts[5120:5120 + _HALF]], axis=0)

  x1 = _tc_layer(seg(x0), x0, Wr1T, Ws1T, b1r)
  x2 = _tc_layer(seg(x1), x1, Wr2T, Ws2T, b2r)
  out = _tc_final(seg(x2), x2, Wr3T, Ws3T, b3r, x1, W1T, W2T, W3T, blr)
  return out[:, :_C]


# speedup vs baseline: 5.7098x; 2.6198x over previous
import jax
import jax.numpy as jnp
from jax import lax
from jax.experimental import pallas as pl
from jax.experimental.pallas import tpu as pltpu
from jax.experimental.pallas import tpu_sc as plsc

_N = 10000
_E = 320000
_D = 128
_C = 40
_NC = 2
_NS = 16
_NW = _NC * _NS
_EPT = _E // _NW
_HALF = _N // _NC
_ACC = 5120
_CAPC = 79
_CAP = _CAPC * 128
_BLK = 1000


def _sc_partition(src_r, dst_r, pad_src, pad_dst):
  mesh = plsc.VectorSubcoreMesh(core_axis_name="c", subcore_axis_name="s")

  def body(src_hbm, dst_hbm, psrc_hbm, pdst_hbm,
           srcp_hbm, dstp_hbm, cnt_hbm,
           srcin, dstin, sb0, sb1, db0, db1, cnt_v):
    c = lax.axis_index("c")
    s = lax.axis_index("s")
    t = c * _NS + s
    pltpu.sync_copy(src_hbm.at[t], srcin)
    pltpu.sync_copy(dst_hbm.at[t], dstin)
    pltpu.sync_copy(psrc_hbm, sb0)
    pltpu.sync_copy(psrc_hbm, sb1)
    pltpu.sync_copy(pdst_hbm, db0)
    pltpu.sync_copy(pdst_hbm, db1)

    iota = lax.iota(jnp.int32, 16)
    pad_d = _HALF + (iota & 63)

    def g16(v, idx):
      return v.at[jnp.clip(idx, 0, 15)].get(mode="promise_in_bounds")

    def merge(bs, bd, pend_s, pend_d, f, wp, vs, vd, cnt):
      sh_s = g16(vs, iota - f)
      sh_d = g16(vd, iota - f)
      in_new = (iota >= f) & (iota < f + cnt)
      m_s = jnp.where(in_new, sh_s, pend_s)
      m_d = jnp.where(in_new, sh_d, pend_d)
      full = (f + cnt) >= 16

      @pl.when(full)
      def _():
        bs[pl.ds(wp * 16, 16)] = m_s
        bd[pl.ds(wp * 16, 16)] = m_d

      rem_n = f + cnt - 16
      r_s = jnp.where(iota < rem_n, g16(vs, iota + (16 - f)), 0)
      r_d = jnp.where(iota < rem_n, g16(vd, iota + (16 - f)), pad_d)
      n_s = jnp.where(full, r_s, m_s)
      n_d = jnp.where(full, r_d, m_d)
      n_f = jnp.where(full, rem_n, f + cnt)
      n_wp = jnp.where(full, wp + 1, wp)
      return n_s, n_d, n_f, n_wp

    def step(i, st):
      ps0, pd0, ps1, pd1, f0, f1, wp0, wp1 = st
      sv = srcin[pl.ds(i * 16, 16)]
      dv = dstin[pl.ds(i * 16, 16)]
      m0 = dv < _HALF
      mi = jnp.where(m0, 1, 0)
      pr = mi
      for k in (1, 2, 4, 8):
        sh = g16(pr, iota - k)
        pr = pr + jnp.where(iota >= k, sh, 0)
      n0 = pr[15]
      excl0 = pr - mi
      r = jnp.where(m0, excl0, n0 + (iota - excl0))
      inv = iota * 0
      for ii in range(16):
        inv = jnp.where(iota == r[ii], ii, inv)
      dvl = jnp.where(m0, dv, dv - _HALF)
      cs = g16(sv, inv)
      cd = g16(dvl, inv)
      ps0, pd0, f0, wp0 = merge(sb0, db0, ps0, pd0, f0, wp0, cs, cd, n0)
      cs1 = g16(cs, iota + n0)
      cd1 = g16(cd, iota + n0)
      ps1, pd1, f1, wp1 = merge(sb1, db1, ps1, pd1, f1, wp1, cs1, cd1,
                                16 - n0)
      return ps0, pd0, ps1, pd1, f0, f1, wp0, wp1

    z = jnp.int32(0)
    init = (iota * 0, pad_d, iota * 0, pad_d, z, z, z, z)
    ps0, pd0, ps1, pd1, f0, f1, wp0, wp1 = lax.fori_loop(
        0, _EPT // 16, step, init)

    sb0[pl.ds(wp0 * 16, 16)] = ps0
    db0[pl.ds(wp0 * 16, 16)] = pd0
    sb1[pl.ds(wp1 * 16, 16)] = ps1
    db1[pl.ds(wp1 * 16, 16)] = pd1
    c0 = wp0 * 16 + f0
    c1 = wp1 * 16 + f1

    cnt_v[...] = jnp.where(iota == 0, c0, 0)
    pltpu.sync_copy(cnt_v, cnt_hbm.at[t])
    cnt_v[...] = jnp.where(iota == 0, c1, 0)
    pltpu.sync_copy(cnt_v, cnt_hbm.at[_NW + t])
    pltpu.sync_copy(sb0, srcp_hbm.at[t])
    pltpu.sync_copy(sb1, srcp_hbm.at[_NW + t])
    pltpu.sync_copy(db0, dstp_hbm.at[t])
    pltpu.sync_copy(db1, dstp_hbm.at[_NW + t])

  f = pl.kernel(
      body,
      out_type=(
          jax.ShapeDtypeStruct((2 * _NW, _CAP), jnp.int32),
          jax.ShapeDtypeStruct((2 * _NW, _CAP), jnp.int32),
          jax.ShapeDtypeStruct((2 * _NW, 16), jnp.int32),
      ),
      mesh=mesh,
      scratch_types=[
          pltpu.VMEM((_EPT,), jnp.int32),
          pltpu.VMEM((_EPT,), jnp.int32),
          pltpu.VMEM((_CAP,), jnp.int32),
          pltpu.VMEM((_CAP,), jnp.int32),
          pltpu.VMEM((_CAP,), jnp.int32),
          pltpu.VMEM((_CAP,), jnp.int32),
          pltpu.VMEM((16,), jnp.int32),
      ],
  )
  return f(src_r, dst_r, pad_src, pad_dst)


def _sc_segment_sum_p(x, srcp, dstp, counts, zeros_blk):
  mesh = plsc.VectorSubcoreMesh(core_axis_name="c", subcore_axis_name="s")

  def body(x_hbm, srcp_hbm, dstp_hbm, cnt_hbm, zer_hbm, out_hbm,
           src_v, dstl_v, stage_v, cv0, cv1, acc_sh, gsem, ssem):
    c = lax.axis_index("c")
    s = lax.axis_index("s")

    pltpu.sync_copy(srcp_hbm.at[c * _NW + 2 * s], src_v.at[pl.ds(0, _CAPC)])
    pltpu.sync_copy(srcp_hbm.at[c * _NW + 2 * s + 1],
                    src_v.at[pl.ds(_CAPC, _CAPC)])
    pltpu.sync_copy(dstp_hbm.at[c * _NW + 2 * s], dstl_v.at[pl.ds(0, _CAPC)])
    pltpu.sync_copy(dstp_hbm.at[c * _NW + 2 * s + 1],
                    dstl_v.at[pl.ds(_CAPC, _CAPC)])
    pltpu.sync_copy(cnt_hbm.at[c * _NW + 2 * s], cv0)
    pltpu.sync_copy(cnt_hbm.at[c * _NW + 2 * s + 1], cv1)

    pltpu.sync_copy(zer_hbm, stage_v.at[pl.ds(0, 128)])
    for k in range(_ACC // 128):
      @pl.when((k % _NS) == s)
      def _():
        pltpu.sync_copy(stage_v.at[pl.ds(0, 128)],
                        acc_sh.at[pl.ds(k * 128, 128)])
    plsc.subcore_barrier()

    cnt0 = cv0[...][0]
    cnt1 = cv1[...][0]
    n0 = (cnt0 + 127) // 128
    n1 = (cnt1 + 127) // 128
    total = n0 + n1
    _NB = 2

    def row_of(j):
      return jnp.where(j < n0, j, j + (_CAPC - n0))

    def stg(p):
      return stage_v.at[pl.ds(p * 128, 128)]

    def gather(j, p):
      return pltpu.async_copy(x_hbm.at[src_v.at[row_of(j)]], stg(p),
                              gsem.at[p])

    def wait_gather(p):
      pltpu.make_async_copy(x_hbm.at[src_v.at[0]], stg(p), gsem.at[p]).wait()

    def scatter(j, p):
      return pltpu.async_copy(stg(p), acc_sh.at[dstl_v.at[row_of(j)]],
                              ssem.at[p], add=True)

    def wait_scatter(p):
      pltpu.make_async_copy(stg(p), acc_sh.at[dstl_v.at[0]],
                            ssem.at[p]).wait()

    def prime(p, carry):
      gather(p, p)
      return carry

    lax.fori_loop(0, jnp.minimum(_NB, total), prime, 0)

    def step(j, carry):
      p = j & 1
      wait_gather(p)
      scatter(j, p)

      @pl.when(j + _NB < total)
      def _():
        wait_scatter(p)
        gather(j + _NB, p)

      return carry

    lax.fori_loop(0, total, step, 0)

    def drain(p, carry):
      wait_scatter(p)
      return carry

    lax.fori_loop(0, jnp.minimum(_NB, total), drain, 0)
    plsc.subcore_barrier()

    for k in range(_ACC // 128):
      @pl.when((k % _NS) == s)
      def _():
        pltpu.sync_copy(acc_sh.at[pl.ds(k * 128, 128)],
                        stage_v.at[pl.ds(0, 128)])
        pltpu.sync_copy(stage_v.at[pl.ds(0, 128)],
                        out_hbm.at[pl.ds(c * _ACC + k * 128, 128)])

  f = pl.kernel(
      body,
      out_type=jax.ShapeDtypeStruct((2 * _ACC, _D), jnp.float32),
      mesh=mesh,
      scratch_types=[
          pltpu.VMEM((2 * _CAPC, 128), jnp.int32),
          pltpu.VMEM((2 * _CAPC, 128), jnp.int32),
          pltpu.VMEM((2 * 128, _D), jnp.float32),
          pltpu.VMEM((16,), jnp.int32),
          pltpu.VMEM((16,), jnp.int32),
          pltpu.VMEM_SHARED((_ACC, _D), jnp.float32),
          pltpu.SemaphoreType.DMA((2,)),
          pltpu.SemaphoreType.DMA((2,)),
      ],
  )
  return f(x, srcp, dstp, counts, zeros_blk)


def _tc_layer(agg, x, WrT, WsT, br):
  def body(a_ref, x_ref, wr_ref, ws_ref, b_ref, o_ref):
    h = jnp.dot(a_ref[...], wr_ref[...], preferred_element_type=jnp.float32)
    h = h + jnp.dot(x_ref[...], ws_ref[...], preferred_element_type=jnp.float32)
    o_ref[...] = jnp.maximum(h + b_ref[...], 0.0)

  return pl.pallas_call(
      body,
      grid=(_N // _BLK,),
      in_specs=[
          pl.BlockSpec((_BLK, _D), lambda i: (i, 0)),
          pl.BlockSpec((_BLK, _D), lambda i: (i, 0)),
          pl.BlockSpec((_D, _D), lambda i: (0, 0)),
          pl.BlockSpec((_D, _D), lambda i: (0, 0)),
          pl.BlockSpec((1, _D), lambda i: (0, 0)),
      ],
      out_specs=pl.BlockSpec((_BLK, _D), lambda i: (i, 0)),
      out_shape=jax.ShapeDtypeStruct((_N, _D), jnp.float32),
  )(agg, x, WrT, WsT, br)


def _tc_final(agg, x2, Wr3T, Ws3T, b3r, x1, W1T, W2T, W3T, blr):
  def body(a_ref, x2_ref, wr_ref, ws_ref, b3_ref,
           x1_ref, w1_ref, w2_ref, w3_ref, bl_ref, o_ref):
    x3 = jnp.dot(a_ref[...], wr_ref[...], preferred_element_type=jnp.float32)
    x3 = x3 + jnp.dot(x2_ref[...], ws_ref[...],
                      preferred_element_type=jnp.float32)
    x3 = jnp.maximum(x3 + b3_ref[...], 0.0)
    logits = jnp.dot(x1_ref[...], w1_ref[...],
                     preferred_element_type=jnp.float32)
    logits = logits + jnp.dot(x2_ref[...], w2_ref[...],
                              preferred_element_type=jnp.float32)
    logits = logits + jnp.dot(x3, w3_ref[...],
                              preferred_element_type=jnp.float32)
    logits = logits + bl_ref[...]
    m = jnp.max(logits, axis=1, keepdims=True)
    z = logits - m
    lse = jnp.log(jnp.sum(jnp.exp(z), axis=1, keepdims=True))
    o_ref[...] = z - lse

  return pl.pallas_call(
      body,
      grid=(_N // _BLK,),
      in_specs=[
          pl.BlockSpec((_BLK, _D), lambda i: (i, 0)),
          pl.BlockSpec((_BLK, _D), lambda i: (i, 0)),
          pl.BlockSpec((_D, _D), lambda i: (0, 0)),
          pl.BlockSpec((_D, _D), lambda i: (0, 0)),
          pl.BlockSpec((1, _D), lambda i: (0, 0)),
          pl.BlockSpec((_BLK, _D), lambda i: (i, 0)),
          pl.BlockSpec((_D, 128), lambda i: (0, 0)),
          pl.BlockSpec((_D, 128), lambda i: (0, 0)),
          pl.BlockSpec((_D, 128), lambda i: (0, 0)),
          pl.BlockSpec((1, 128), lambda i: (0, 0)),
      ],
      out_specs=pl.BlockSpec((_BLK, 128), lambda i: (i, 0)),
      out_shape=jax.ShapeDtypeStruct((_N, 128), jnp.float32),
  )(agg, x2, Wr3T, Ws3T, b3r, x1, W1T, W2T, W3T, blr)


def kernel(x0, edge_index, Wr1, Ws1, b1, Wr2, Ws2, b2, Wr3, Ws3, b3, Wl, bl):
  src_r = edge_index[0].reshape(_NW, _EPT)
  dst_r = edge_index[1].reshape(_NW, _EPT)
  pad_src = jnp.zeros((_CAP,), jnp.int32)
  pad_dst = _HALF + (jnp.arange(_CAP, dtype=jnp.int32) & 63)
  zeros_blk = jnp.zeros((128, _D), jnp.float32)

  Wr1T, Ws1T = Wr1.T, Ws1.T
  Wr2T, Ws2T = Wr2.T, Ws2.T
  Wr3T, Ws3T = Wr3.T, Ws3.T
  b1r = b1.reshape(1, _D)
  b2r = b2.reshape(1, _D)
  b3r = b3.reshape(1, _D)
  WlTp = jnp.pad(Wl.T, ((0, 0), (0, 128 - _C)))
  W1T, W2T, W3T = WlTp[:_D], WlTp[_D:2 * _D], WlTp[2 * _D:]
  blr = jnp.pad(bl, (0, 128 - _C), constant_values=-1e30).reshape(1, 128)

  srcp, dstp, counts = _sc_partition(src_r, dst_r, pad_src, pad_dst)
  srcp = srcp.reshape(2 * _NW, _CAPC, 128)
  dstp = dstp.reshape(2 * _NW, _CAPC, 128)

  def seg(x):
    parts = _sc_segment_sum_p(x, srcp, dstp, counts, zeros_blk)
    return jnp.concatenate([parts[:_HALF], parts[_ACC:_ACC + _HALF]], axis=0)

  x1 = _tc_layer(seg(x0), x0, Wr1T, Ws1T, b1r)
  x2 = _tc_layer(seg(x1), x1, Wr2T, Ws2T, b2r)
  out = _tc_final(seg(x2), x2, Wr3T, Ws3T, b3r, x1, W1T, W2T, W3T, blr)
  return out[:, :_C]


# speedup vs baseline: 5.9711x; 1.0458x over previous
import jax
import jax.numpy as jnp
from jax import lax
from jax.experimental import pallas as pl
from jax.experimental.pallas import tpu as pltpu
from jax.experimental.pallas import tpu_sc as plsc

_N = 10000
_E = 320000
_D = 128
_C = 40
_NC = 2
_NS = 16
_NW = _NC * _NS
_EPT = _E // _NW
_HALF = _N // _NC
_ACC = 5064
_CAPC = 79
_CAP = _CAPC * 128
_BLK = 1000


def _sc_partition(src_r, dst_r, pad_src, pad_dst):
  mesh = plsc.VectorSubcoreMesh(core_axis_name="c", subcore_axis_name="s")

  def body(src_hbm, dst_hbm, psrc_hbm, pdst_hbm,
           srcp_hbm, dstp_hbm, cnt_hbm,
           srcin, dstin, sb0, sb1, db0, db1, cnt_v):
    c = lax.axis_index("c")
    s = lax.axis_index("s")
    t = c * _NS + s
    pltpu.sync_copy(src_hbm.at[t], srcin)
    pltpu.sync_copy(dst_hbm.at[t], dstin)
    pltpu.sync_copy(psrc_hbm, sb0)
    pltpu.sync_copy(psrc_hbm, sb1)
    pltpu.sync_copy(pdst_hbm, db0)
    pltpu.sync_copy(pdst_hbm, db1)

    iota = lax.iota(jnp.int32, 16)
    pad_d = _HALF + (iota & 63)

    def g16(v, idx):
      return v.at[jnp.clip(idx, 0, 15)].get(mode="promise_in_bounds")

    def merge(bs, bd, pend_s, pend_d, f, wp, vs, vd, cnt):
      sh_s = g16(vs, iota - f)
      sh_d = g16(vd, iota - f)
      in_new = (iota >= f) & (iota < f + cnt)
      m_s = jnp.where(in_new, sh_s, pend_s)
      m_d = jnp.where(in_new, sh_d, pend_d)
      full = (f + cnt) >= 16

      @pl.when(full)
      def _():
        bs[pl.ds(wp * 16, 16)] = m_s
        bd[pl.ds(wp * 16, 16)] = m_d

      rem_n = f + cnt - 16
      r_s = jnp.where(iota < rem_n, g16(vs, iota + (16 - f)), 0)
      r_d = jnp.where(iota < rem_n, g16(vd, iota + (16 - f)), pad_d)
      n_s = jnp.where(full, r_s, m_s)
      n_d = jnp.where(full, r_d, m_d)
      n_f = jnp.where(full, rem_n, f + cnt)
      n_wp = jnp.where(full, wp + 1, wp)
      return n_s, n_d, n_f, n_wp

    def step(i, st):
      ps0, pd0, ps1, pd1, f0, f1, wp0, wp1 = st
      sv = srcin[pl.ds(i * 16, 16)]
      dv = dstin[pl.ds(i * 16, 16)]
      m0 = dv < _HALF
      mi = jnp.where(m0, 1, 0)
      pr = mi
      for k in (1, 2, 4, 8):
        sh = g16(pr, iota - k)
        pr = pr + jnp.where(iota >= k, sh, 0)
      n0 = pr[15]
      excl0 = pr - mi
      r = jnp.where(m0, excl0, n0 + (iota - excl0))
      inv = iota * 0
      for ii in range(16):
        inv = jnp.where(iota == r[ii], ii, inv)
      dvl = jnp.where(m0, dv, dv - _HALF)
      cs = g16(sv, inv)
      cd = g16(dvl, inv)
      ps0, pd0, f0, wp0 = merge(sb0, db0, ps0, pd0, f0, wp0, cs, cd, n0)
      cs1 = g16(cs, iota + n0)
      cd1 = g16(cd, iota + n0)
      ps1, pd1, f1, wp1 = merge(sb1, db1, ps1, pd1, f1, wp1, cs1, cd1,
                                16 - n0)
      return ps0, pd0, ps1, pd1, f0, f1, wp0, wp1

    z = jnp.int32(0)
    init = (iota * 0, pad_d, iota * 0, pad_d, z, z, z, z)
    ps0, pd0, ps1, pd1, f0, f1, wp0, wp1 = lax.fori_loop(
        0, _EPT // 16, step, init)

    sb0[pl.ds(wp0 * 16, 16)] = ps0
    db0[pl.ds(wp0 * 16, 16)] = pd0
    sb1[pl.ds(wp1 * 16, 16)] = ps1
    db1[pl.ds(wp1 * 16, 16)] = pd1
    c0 = wp0 * 16 + f0
    c1 = wp1 * 16 + f1

    cnt_v[...] = jnp.where(iota == 0, c0, 0)
    pltpu.sync_copy(cnt_v, cnt_hbm.at[t])
    cnt_v[...] = jnp.where(iota == 0, c1, 0)
    pltpu.sync_copy(cnt_v, cnt_hbm.at[_NW + t])
    pltpu.sync_copy(sb0, srcp_hbm.at[t])
    pltpu.sync_copy(sb1, srcp_hbm.at[_NW + t])
    pltpu.sync_copy(db0, dstp_hbm.at[t])
    pltpu.sync_copy(db1, dstp_hbm.at[_NW + t])

  f = pl.kernel(
      body,
      out_type=(
          jax.ShapeDtypeStruct((2 * _NW, _CAP), jnp.int32),
          jax.ShapeDtypeStruct((2 * _NW, _CAP), jnp.int32),
          jax.ShapeDtypeStruct((2 * _NW, 16), jnp.int32),
      ),
      mesh=mesh,
      scratch_types=[
          pltpu.VMEM((_EPT,), jnp.int32),
          pltpu.VMEM((_EPT,), jnp.int32),
          pltpu.VMEM((_CAP,), jnp.int32),
          pltpu.VMEM((_CAP,), jnp.int32),
          pltpu.VMEM((_CAP,), jnp.int32),
          pltpu.VMEM((_CAP,), jnp.int32),
          pltpu.VMEM((16,), jnp.int32),
      ],
  )
  return f(src_r, dst_r, pad_src, pad_dst)


def _sc_segment_sum_p(x, srcp, dstp, counts, zeros_blk):
  mesh = plsc.VectorSubcoreMesh(core_axis_name="c", subcore_axis_name="s")

  def body(x_hbm, srcp_hbm, dstp_hbm, cnt_hbm, zer_hbm, out_hbm,
           src_v, dstl_v, stage_v, cv0, cv1, acc_sh, gsem, ssem):
    c = lax.axis_index("c")
    s = lax.axis_index("s")

    pltpu.sync_copy(srcp_hbm.at[c * _NW + 2 * s], src_v.at[pl.ds(0, _CAPC)])
    pltpu.sync_copy(srcp_hbm.at[c * _NW + 2 * s + 1],
                    src_v.at[pl.ds(_CAPC, _CAPC)])
    pltpu.sync_copy(dstp_hbm.at[c * _NW + 2 * s], dstl_v.at[pl.ds(0, _CAPC)])
    pltpu.sync_copy(dstp_hbm.at[c * _NW + 2 * s + 1],
                    dstl_v.at[pl.ds(_CAPC, _CAPC)])
    pltpu.sync_copy(cnt_hbm.at[c * _NW + 2 * s], cv0)
    pltpu.sync_copy(cnt_hbm.at[c * _NW + 2 * s + 1], cv1)

    cnt0 = cv0[...][0]
    cnt1 = cv1[...][0]
    n0 = (cnt0 + 127) // 128
    n1 = (cnt1 + 127) // 128
    total = n0 + n1
    _NB = 3

    def row_of(j):
      return jnp.where(j < n0, j, j + (_CAPC - n0))

    def stg(p):
      return stage_v.at[pl.ds(p * 128, 128)]

    def gather(j, p):
      return pltpu.async_copy(x_hbm.at[src_v.at[row_of(j)]], stg(p),
                              gsem.at[p])

    def wait_gather(p):
      pltpu.make_async_copy(x_hbm.at[src_v.at[0]], stg(p), gsem.at[p]).wait()

    def scatter(j, p):
      return pltpu.async_copy(stg(p), acc_sh.at[dstl_v.at[row_of(j)]],
                              ssem.at[p], add=True)

    def wait_scatter(p):
      pltpu.make_async_copy(stg(p), acc_sh.at[dstl_v.at[0]],
                            ssem.at[p]).wait()

    def prime(p, carry):
      gather(p, p)
      return carry

    lax.fori_loop(0, jnp.minimum(_NB, total), prime, 0)

    for k in range(39):
      @pl.when((k % _NS) == s)
      def _():
        pltpu.sync_copy(zer_hbm, acc_sh.at[pl.ds(k * 128, 128)])

    @pl.when(s == 15)
    def _():
      pltpu.sync_copy(zer_hbm.at[pl.ds(0, 8)], acc_sh.at[pl.ds(4992, 8)])
    plsc.subcore_barrier()

    def step(j, carry):
      p = lax.rem(j, _NB)
      wait_gather(p)
      scatter(j, p)

      @pl.when(j + _NB < total)
      def _():
        wait_scatter(p)
        gather(j + _NB, p)

      return carry

    lax.fori_loop(0, total, step, 0)

    def drain(p, carry):
      wait_scatter(p)
      return carry

    lax.fori_loop(0, jnp.minimum(_NB, total), drain, 0)
    plsc.subcore_barrier()

    for k in range(39):
      @pl.when((k % _NS) == s)
      def _():
        pltpu.sync_copy(acc_sh.at[pl.ds(k * 128, 128)],
                        stage_v.at[pl.ds(0, 128)])
        pltpu.sync_copy(stage_v.at[pl.ds(0, 128)],
                        out_hbm.at[pl.ds(c * _HALF + k * 128, 128)])

    @pl.when(s == 15)
    def _():
      pltpu.sync_copy(acc_sh.at[pl.ds(4992, 8)], stage_v.at[pl.ds(128, 8)])
      pltpu.sync_copy(stage_v.at[pl.ds(128, 8)],
                      out_hbm.at[pl.ds(c * _HALF + 4992, 8)])

  f = pl.kernel(
      body,
      out_type=jax.ShapeDtypeStruct((_N, _D), jnp.float32),
      mesh=mesh,
      scratch_types=[
          pltpu.VMEM((2 * _CAPC, 128), jnp.int32),
          pltpu.VMEM((2 * _CAPC, 128), jnp.int32),
          pltpu.VMEM((3 * 128, _D), jnp.float32),
          pltpu.VMEM((16,), jnp.int32),
          pltpu.VMEM((16,), jnp.int32),
          pltpu.VMEM_SHARED((_ACC, _D), jnp.float32),
          pltpu.SemaphoreType.DMA((3,)),
          pltpu.SemaphoreType.DMA((3,)),
      ],
  )
  return f(x, srcp, dstp, counts, zeros_blk)


def _tc_layer(agg, x, WrT, WsT, br):
  def body(a_ref, x_ref, wr_ref, ws_ref, b_ref, o_ref):
    h = jnp.dot(a_ref[...], wr_ref[...], preferred_element_type=jnp.float32)
    h = h + jnp.dot(x_ref[...], ws_ref[...], preferred_element_type=jnp.float32)
    o_ref[...] = jnp.maximum(h + b_ref[...], 0.0)

  return pl.pallas_call(
      body,
      grid=(_N // _BLK,),
      in_specs=[
          pl.BlockSpec((_BLK, _D), lambda i: (i, 0)),
          pl.BlockSpec((_BLK, _D), lambda i: (i, 0)),
          pl.BlockSpec((_D, _D), lambda i: (0, 0)),
          pl.BlockSpec((_D, _D), lambda i: (0, 0)),
          pl.BlockSpec((1, _D), lambda i: (0, 0)),
      ],
      out_specs=pl.BlockSpec((_BLK, _D), lambda i: (i, 0)),
      out_shape=jax.ShapeDtypeStruct((_N, _D), jnp.float32),
  )(agg, x, WrT, WsT, br)


def _tc_final(agg, x2, Wr3T, Ws3T, b3r, x1, W1T, W2T, W3T, blr):
  def body(a_ref, x2_ref, wr_ref, ws_ref, b3_ref,
           x1_ref, w1_ref, w2_ref, w3_ref, bl_ref, o_ref):
    x3 = jnp.dot(a_ref[...], wr_ref[...], preferred_element_type=jnp.float32)
    x3 = x3 + jnp.dot(x2_ref[...], ws_ref[...],
                      preferred_element_type=jnp.float32)
    x3 = jnp.maximum(x3 + b3_ref[...], 0.0)
    logits = jnp.dot(x1_ref[...], w1_ref[...],
                     preferred_element_type=jnp.float32)
    logits = logits + jnp.dot(x2_ref[...], w2_ref[...],
                              preferred_element_type=jnp.float32)
    logits = logits + jnp.dot(x3, w3_ref[...],
                              preferred_element_type=jnp.float32)
    logits = logits + bl_ref[...]
    m = jnp.max(logits, axis=1, keepdims=True)
    z = logits - m
    lse = jnp.log(jnp.sum(jnp.exp(z), axis=1, keepdims=True))
    o_ref[...] = z - lse

  return pl.pallas_call(
      body,
      grid=(_N // _BLK,),
      in_specs=[
          pl.BlockSpec((_BLK, _D), lambda i: (i, 0)),
          pl.BlockSpec((_BLK, _D), lambda i: (i, 0)),
          pl.BlockSpec((_D, _D), lambda i: (0, 0)),
          pl.BlockSpec((_D, _D), lambda i: (0, 0)),
          pl.BlockSpec((1, _D), lambda i: (0, 0)),
          pl.BlockSpec((_BLK, _D), lambda i: (i, 0)),
          pl.BlockSpec((_D, 128), lambda i: (0, 0)),
          pl.BlockSpec((_D, 128), lambda i: (0, 0)),
          pl.BlockSpec((_D, 128), lambda i: (0, 0)),
          pl.BlockSpec((1, 128), lambda i: (0, 0)),
      ],
      out_specs=pl.BlockSpec((_BLK, 128), lambda i: (i, 0)),
      out_shape=jax.ShapeDtypeStruct((_N, 128), jnp.float32),
  )(agg, x2, Wr3T, Ws3T, b3r, x1, W1T, W2T, W3T, blr)


def kernel(x0, edge_index, Wr1, Ws1, b1, Wr2, Ws2, b2, Wr3, Ws3, b3, Wl, bl):
  src_r = edge_index[0].reshape(_NW, _EPT)
  dst_r = edge_index[1].reshape(_NW, _EPT)
  pad_src = jnp.zeros((_CAP,), jnp.int32)
  pad_dst = _HALF + (jnp.arange(_CAP, dtype=jnp.int32) & 63)
  zeros_blk = jnp.zeros((128, _D), jnp.float32)

  Wr1T, Ws1T = Wr1.T, Ws1.T
  Wr2T, Ws2T = Wr2.T, Ws2.T
  Wr3T, Ws3T = Wr3.T, Ws3.T
  b1r = b1.reshape(1, _D)
  b2r = b2.reshape(1, _D)
  b3r = b3.reshape(1, _D)
  WlTp = jnp.pad(Wl.T, ((0, 0), (0, 128 - _C)))
  W1T, W2T, W3T = WlTp[:_D], WlTp[_D:2 * _D], WlTp[2 * _D:]
  blr = jnp.pad(bl, (0, 128 - _C), constant_values=-1e30).reshape(1, 128)

  srcp, dstp, counts = _sc_partition(src_r, dst_r, pad_src, pad_dst)
  srcp = srcp.reshape(2 * _NW, _CAPC, 128)
  dstp = dstp.reshape(2 * _NW, _CAPC, 128)

  def seg(x):
    return _sc_segment_sum_p(x, srcp, dstp, counts, zeros_blk)

  x1 = _tc_layer(seg(x0), x0, Wr1T, Ws1T, b1r)
  x2 = _tc_layer(seg(x1), x1, Wr2T, Ws2T, b2r)
  out = _tc_final(seg(x2), x2, Wr3T, Ws3T, b3r, x1, W1T, W2T, W3T, blr)
  return out[:, :_C]
